# baseline (speedup 1.0000x reference)
import numpy as np

import jax
import jax.numpy as jnp
from jax import lax
from jax.experimental import pallas as pl
from jax.experimental.pallas import tpu as pltpu
from jax.experimental.pallas import tpu_sc as plsc

NUM_CLASS = 100000
LOW_DIM = 128
ROWS = 3 * NUM_CLASS

NC = 2
NS = 16
NW = NC * NS
LANES = 16
VPR = LOW_DIM // LANES

CHUNK = 240
DEPTH = 2
NCHUNKS = ROWS // CHUNK
assert NCHUNKS * CHUNK == ROWS

_MAGIC = np.int32(0x5F3759DF)


def _normalize_row(buf, r):
    vs = [buf[r, pl.ds(LANES * k, LANES)] for k in range(VPR)]
    ss = vs[0] * vs[0]
    for k in range(1, VPR):
        ss = ss + vs[k] * vs[k]
    tot = jnp.sum(ss)
    i = lax.bitcast_convert_type(tot, jnp.int32)
    y = lax.bitcast_convert_type(_MAGIC - lax.shift_right_logical(i, 1),
                                 jnp.float32)
    h = tot * 0.5
    for _ in range(2):
        y = y * (1.5 - h * y * y)
    norm = tot * y
    scale_s = jnp.where(norm > 1e-12, y, jnp.float32(1e12))
    scale = lax.broadcast(scale_s, (LANES,))
    for k in range(VPR):
        buf[r, pl.ds(LANES * k, LANES)] = vs[k] * scale


def _body(feat_hbm, r0_hbm, proto_hbm, out_hbm, buf2, spmem, feat_v, r0_v,
          lsem, hsem, ssem):
    wid = lax.axis_index("s") * NC + lax.axis_index("c")
    pltpu.sync_copy(feat_hbm, feat_v)
    pltpu.sync_copy(r0_hbm, r0_v)
    r0 = r0_v[...][0]

    n = (NCHUNKS - wid + NW - 1) // NW

    def base_of(t):
        return (wid + NW * t) * CHUNK

    def start_load(b, t):
        pltpu.make_async_copy(proto_hbm.at[pl.ds(base_of(t), CHUNK)],
                              buf2.at[b], lsem.at[b]).start()

    def wait_load(b):
        pltpu.make_async_copy(proto_hbm.at[pl.ds(0, CHUNK)],
                              buf2.at[b], lsem.at[b]).wait()

    sid = lax.axis_index("s")

    def start_hop(b):
        pltpu.make_async_copy(buf2.at[b], spmem.at[sid, b],
                              hsem.at[b]).start()

    def wait_hop(b):
        pltpu.make_async_copy(buf2.at[b], spmem.at[sid, b],
                              hsem.at[b]).wait()

    def start_hbm_store(b, t):
        pltpu.make_async_copy(spmem.at[sid, b],
                              out_hbm.at[pl.ds(base_of(t), CHUNK)],
                              ssem.at[b]).start()

    def wait_hbm_store(b):
        pltpu.make_async_copy(spmem.at[sid, b],
                              out_hbm.at[pl.ds(0, CHUNK)],
                              ssem.at[b]).wait()

    def process(b, t):
        buf = buf2.at[b]
        base = base_of(t)
        for k in range(2):
            off = (r0 + k) - base

            @pl.when((off >= 0) & (off < CHUNK))
            def _():
                for j in range(VPR):
                    buf[off, pl.ds(LANES * j, LANES)] = (
                        feat_v[k, pl.ds(LANES * j, LANES)])

        @plsc.parallel_loop(0, CHUNK, unroll=2)
        def _(r):
            _normalize_row(buf, r)

    start_load(0, 0)

    def body(i, carry):
        for b in (0, 1):
            t = 2 * i + b

            @pl.when(t < n)
            def _(b=b, t=t):
                wait_load(b)

                @pl.when(t >= 1)
                def _():
                    wait_hop(1 - b)
                    start_hbm_store(1 - b, t - 1)

                @pl.when(t + 1 < n)
                def _():
                    start_load(1 - b, t + 1)

                process(b, t)

                @pl.when(t >= 2)
                def _():
                    wait_hbm_store(b)

                start_hop(b)
        return carry

    lax.fori_loop(0, (n + 1) // 2, body, 0)
    last = n - 1
    lb = last % 2
    wait_hop(lb)
    start_hbm_store(lb, last)
    wait_hbm_store(1 - lb)
    wait_hbm_store(lb)


@jax.jit
def _sc_normalize(feat, r0vec, prototypes):
    mesh = plsc.VectorSubcoreMesh(core_axis_name="c", subcore_axis_name="s")
    return pl.kernel(
        _body,
        out_type=jax.ShapeDtypeStruct((ROWS, LOW_DIM), jnp.float32),
        mesh=mesh,
        compiler_params=pltpu.CompilerParams(needs_layout_passes=False),
        scratch_types=[
            pltpu.VMEM((DEPTH, CHUNK, LOW_DIM), jnp.float32),
            pltpu.VMEM_SHARED((NS, DEPTH, CHUNK, LOW_DIM), jnp.float32),
            pltpu.VMEM((2, LOW_DIM), jnp.float32),
            pltpu.VMEM((LANES,), jnp.int32),
            pltpu.SemaphoreType.DMA((DEPTH,)),
            pltpu.SemaphoreType.DMA((DEPTH,)),
            pltpu.SemaphoreType.DMA((DEPTH,)),
        ],
    )(feat, r0vec, prototypes)


def kernel(feat, label, prototypes):
    r0vec = jnp.full((LANES,), 3 * label, dtype=jnp.int32)
    return _sc_normalize(feat, r0vec, prototypes)


# speedup vs baseline: 1.0316x; 1.0316x over previous
import numpy as np

import jax
import jax.numpy as jnp
from jax import lax
from jax.experimental import pallas as pl
from jax.experimental.pallas import tpu as pltpu
from jax.experimental.pallas import tpu_sc as plsc

NUM_CLASS = 100000
LOW_DIM = 128
ROWS = 3 * NUM_CLASS

NC = 2
NS = 16
NW = NC * NS
LANES = 16
VPR = LOW_DIM // LANES

CHUNK = 480
DEPTH = 2
NCHUNKS = ROWS // CHUNK
assert NCHUNKS * CHUNK == ROWS

_MAGIC = np.int32(0x5F3759DF)


def _normalize_row(buf, r):
    vs = [buf[r, pl.ds(LANES * k, LANES)] for k in range(VPR)]
    ss = vs[0] * vs[0]
    for k in range(1, VPR):
        ss = ss + vs[k] * vs[k]
    tot = jnp.sum(ss)
    i = lax.bitcast_convert_type(tot, jnp.int32)
    y = lax.bitcast_convert_type(_MAGIC - lax.shift_right_logical(i, 1),
                                 jnp.float32)
    h = tot * 0.5
    for _ in range(2):
        y = y * (1.5 - h * y * y)
    norm = tot * y
    scale_s = jnp.where(norm > 1e-12, y, jnp.float32(1e12))
    scale = lax.broadcast(scale_s, (LANES,))
    for k in range(VPR):
        buf[r, pl.ds(LANES * k, LANES)] = vs[k] * scale


def _body(feat_hbm, r0_hbm, proto_hbm, out_hbm, buf2, feat_v, r0_v,
          lsem, ssem):
    wid = lax.axis_index("s") * NC + lax.axis_index("c")
    pltpu.sync_copy(feat_hbm, feat_v)
    pltpu.sync_copy(r0_hbm, r0_v)
    r0 = r0_v[...][0]

    n = (NCHUNKS - wid + NW - 1) // NW

    def base_of(t):
        return (wid + NW * t) * CHUNK

    def start_load(b, t):
        pltpu.make_async_copy(proto_hbm.at[pl.ds(base_of(t), CHUNK)],
                              buf2.at[b], lsem.at[b]).start()

    def wait_load(b):
        pltpu.make_async_copy(proto_hbm.at[pl.ds(0, CHUNK)],
                              buf2.at[b], lsem.at[b]).wait()

    def start_store(b, t):
        pltpu.make_async_copy(buf2.at[b],
                              out_hbm.at[pl.ds(base_of(t), CHUNK)],
                              ssem.at[b]).start()

    def wait_store(b):
        pltpu.make_async_copy(buf2.at[b], out_hbm.at[pl.ds(0, CHUNK)],
                              ssem.at[b]).wait()

    def process(b, t):
        buf = buf2.at[b]
        base = base_of(t)
        for k in range(2):
            off = (r0 + k) - base

            @pl.when((off >= 0) & (off < CHUNK))
            def _():
                for j in range(VPR):
                    buf[off, pl.ds(LANES * j, LANES)] = (
                        feat_v[k, pl.ds(LANES * j, LANES)])

        @plsc.parallel_loop(0, CHUNK, unroll=2)
        def _(r):
            _normalize_row(buf, r)

    for b in range(DEPTH - 1):
        start_load(b, b)

    def blk(i, carry):
        for k in range(DEPTH):
            t = DEPTH * i + k

            @pl.when(t < n)
            def _(b=k, t=t):
                wait_load(b)
                pre = t + DEPTH - 1

                @pl.when(pre < n)
                def _():
                    @pl.when(t >= 1)
                    def _():
                        wait_store((t - 1) % DEPTH)
                    start_load(pre % DEPTH, pre)

                process(b, t)
                start_store(b, t)
        return carry

    lax.fori_loop(0, (n + DEPTH - 1) // DEPTH, blk, 0)
    for b in range(DEPTH):
        wait_store(b)


@jax.jit
def _sc_normalize(feat, r0vec, prototypes):
    mesh = plsc.VectorSubcoreMesh(core_axis_name="c", subcore_axis_name="s")
    return pl.kernel(
        _body,
        out_type=jax.ShapeDtypeStruct((ROWS, LOW_DIM), jnp.float32),
        mesh=mesh,
        compiler_params=pltpu.CompilerParams(needs_layout_passes=False),
        scratch_types=[
            pltpu.VMEM((DEPTH, CHUNK, LOW_DIM), jnp.float32),
            pltpu.VMEM((2, LOW_DIM), jnp.float32),
            pltpu.VMEM((LANES,), jnp.int32),
            pltpu.SemaphoreType.DMA((DEPTH,)),
            pltpu.SemaphoreType.DMA((DEPTH,)),
        ],
    )(feat, r0vec, prototypes)


def kernel(feat, label, prototypes):
    r0vec = jnp.full((LANES,), 3 * label, dtype=jnp.int32)
    return _sc_normalize(feat, r0vec, prototypes)
